# Initial kernel scaffold; baseline (speedup 1.0000x reference)
import jax
import jax.numpy as jnp
from jax.experimental import pallas as pl


def kernel(feat, table):
    raise NotImplementedError("write your pallas kernel here")


# speedup vs baseline: 1.6910x; 1.6910x over previous
import functools

import jax
import jax.numpy as jnp
from jax import lax
from jax.experimental import pallas as pl
from jax.experimental.pallas import tpu as pltpu
from jax.experimental.pallas import tpu_sc as plsc

_NUM_CLASSES = 100000
_MULT = 3
_EMBED = 64

_B = 4096 * 200 * _MULT
_NC = 2
_NS = 16
_NW = _NC * _NS
_PER_W = _B // _NW
_G = 768
_NGRP = _PER_W // _G
_NT = _G // 128


def _sc_gather(feat_flat, table):
  mesh = plsc.VectorSubcoreMesh(core_axis_name="c", subcore_axis_name="s")

  @functools.partial(
      pl.kernel,
      mesh=mesh,
      out_type=jax.ShapeDtypeStruct((_B, _EMBED), jnp.float32),
      scratch_types=[
          pltpu.VMEM((_G,), jnp.int32),
          pltpu.VMEM((_NT, 128), jnp.int32),
          pltpu.VMEM((_G, _EMBED), jnp.float32),
          pltpu.SemaphoreType.DMA,
      ],
      compiler_params=pltpu.CompilerParams(use_tc_tiling_on_sc=False),
  )
  def k(feat_hbm, table_hbm, out_hbm, feat_v, idx_v, rows_v, sem):
    wid = lax.axis_index("s") * _NC + lax.axis_index("c")
    lane = lax.iota(jnp.int32, 16)

    def body(g, carry):
      base = wid * _PER_W + g * _G
      pltpu.sync_copy(feat_hbm.at[pl.ds(base, _G)], feat_v)
      for s in range(_G // 16):
        off = ((lane + (s * 16) % 3) % 3) * _NUM_CLASSES
        idx_v[s // 8, pl.ds((s % 8) * 16, 16)] = (
            feat_v[pl.ds(s * 16, 16)] + off)
      copies = [
          pltpu.async_copy(
              table_hbm.at[idx_v.at[t]],
              rows_v.at[pl.ds(t * 128, 128)],
              sem,
          )
          for t in range(_NT)
      ]
      for c in copies:
        c.wait()
      pltpu.sync_copy(rows_v, out_hbm.at[pl.ds(base, _G)])
      return carry

    lax.fori_loop(0, _NGRP, body, 0)

  return k(feat_flat, table)


def kernel(feat, table):
  feat_flat = feat.reshape(-1)
  out = _sc_gather(feat_flat, table)
  return out.reshape(feat.shape + (_EMBED,))
